# Initial kernel scaffold; baseline (speedup 1.0000x reference)
import jax
import jax.numpy as jnp
from jax.experimental import pallas as pl


def kernel(x, edge_index, W1, W2, W3):
    raise NotImplementedError("write your pallas kernel here")


# speedup vs baseline: 7.1346x; 7.1346x over previous
import functools

import jax
import jax.numpy as jnp
from jax import lax
from jax.experimental import pallas as pl
from jax.experimental.pallas import tpu as pltpu
from jax.experimental.pallas import tpu_sc as plsc

NC = 2
NS = 16
CHUNK = 128
BLK = 4


def _cdiv(a, b):
    return (a + b - 1) // b


def _sc_deg(dst2d, n_nodes):
    n_chunks = dst2d.shape[0]
    ct = n_chunks // (NC * NS)
    hr = _cdiv(n_nodes + 1, 128)

    mesh = plsc.VectorSubcoreMesh(core_axis_name="c", subcore_axis_name="s", num_cores=NC, num_subcores=NS)

    @functools.partial(
        pl.kernel,
        out_type=jax.ShapeDtypeStruct((NC * NS, hr, 128), jnp.float32),
        mesh=mesh,
        compiler_params=pltpu.CompilerParams(needs_layout_passes=False),
        scratch_types=[
            pltpu.VMEM((hr, 128), jnp.float32),
            pltpu.VMEM((ct, CHUNK), jnp.int32),
        ],
    )
    def deg_kernel(dst_hbm, deg_out, hist, dbuf):
        c = lax.axis_index("c")
        s = lax.axis_index("s")
        wid = s * NC + c

        def zero_body(i, _):
            hist[i // 8, pl.ds((i % 8) * 16, 16)] = jnp.zeros((16,), jnp.float32)
            return _

        lax.fori_loop(0, hr * 8, zero_body, 0)

        pltpu.sync_copy(dst_hbm.at[pl.ds(wid * ct, ct)], dbuf)

        ones = jnp.ones((16,), jnp.float32)

        def acc_body(i, _):
            row = i // (CHUNK // 16)
            grp = i % (CHUNK // 16)
            idx = dbuf[row, pl.ds(grp * 16, 16)]
            plsc.addupdate_scatter(
                hist,
                [lax.shift_right_logical(idx, 7), lax.bitwise_and(idx, 127)],
                ones)
            return _

        lax.fori_loop(0, ct * (CHUNK // 16), acc_body, 0)

        pltpu.sync_copy(hist, deg_out.at[wid])

    return deg_kernel(dst2d)


def _sc_scatter(h_stacked, src2, dst2d, n_nodes):
    n_chunks = dst2d.shape[0]
    ct = n_chunks // NS
    iblk = 8
    wave = 2
    nblk = ct // iblk
    acc_r = _cdiv(n_nodes + 8, NS * 128) * NS * 128
    zr = acc_r // NS
    wr_a = _cdiv(n_nodes // NS, 8) * 8
    wr_b = n_nodes - (NS - 1) * wr_a
    zb = 128

    mesh = plsc.VectorSubcoreMesh(core_axis_name="c", subcore_axis_name="s", num_cores=NC, num_subcores=NS)

    @functools.partial(
        pl.kernel,
        out_type=jax.ShapeDtypeStruct((NC * n_nodes, 128), jnp.float32),
        mesh=mesh,
        scratch_types=[
            pltpu.VMEM_SHARED((acc_r, 128), jnp.float32),
            pltpu.VMEM((wave * CHUNK, 128), jnp.float32),
            pltpu.VMEM((iblk, CHUNK), jnp.int32),
            pltpu.VMEM((iblk, CHUNK), jnp.int32),
            pltpu.SemaphoreType.DMA,
        ],
    )
    def scat_kernel(h_hbm, src_hbm, dst_hbm, out_hbm,
                    acc, rowbuf, sidx, didx, sem):
        c = lax.axis_index("c")
        s = lax.axis_index("s")

        def zb_body(i, _):
            rowbuf[i // 8, pl.ds((i % 8) * 16, 16)] = jnp.zeros((16,), jnp.float32)
            return _

        lax.fori_loop(0, zb * 8, zb_body, 0)

        base = pl.multiple_of(s * zr, 8)
        for k in range(zr // zb):
            pltpu.sync_copy(rowbuf.at[pl.ds(0, zb)], acc.at[pl.ds(base + k * zb, zb)])
        plsc.subcore_barrier()

        def blk_body(b, _):
            ch0 = pl.multiple_of(s * ct + b * iblk, 8)
            pltpu.sync_copy(src_hbm.at[c, pl.ds(ch0, iblk)], sidx)
            pltpu.sync_copy(dst_hbm.at[pl.ds(ch0, iblk)], didx)
            for w in range(iblk // wave):
                cps = [
                    pltpu.async_copy(h_hbm.at[sidx.at[w * wave + j]],
                                     rowbuf.at[pl.ds(j * CHUNK, CHUNK)], sem)
                    for j in range(wave)
                ]
                for cp in cps:
                    cp.wait()
                for j in range(wave):
                    pltpu.sync_copy(rowbuf.at[pl.ds(j * CHUNK, CHUNK)],
                                    acc.at[didx.at[w * wave + j]], add=True)
            return _

        lax.fori_loop(0, nblk, blk_body, 0)

        plsc.subcore_barrier()
        rd = pl.multiple_of(s * wr_a, 8)
        wo = pl.multiple_of(c * n_nodes + s * wr_a, 8)

        @pl.when(s < NS - 1)
        def _():
            pltpu.sync_copy(acc.at[pl.ds(rd, wr_a)], out_hbm.at[pl.ds(wo, wr_a)])

        @pl.when(s == NS - 1)
        def _():
            pltpu.sync_copy(acc.at[pl.ds(rd, wr_b)], out_hbm.at[pl.ds(wo, wr_b)])

    return scat_kernel(h_stacked, src2, dst2d)


def _tc_dinv(degp, n):
    w = degp.shape[1]

    def body(degp_ref, o_ref):
        deg = jnp.sum(degp_ref[...], axis=0)[:n]
        o_ref[...] = lax.rsqrt(1.0 + deg)[:, None]

    return pl.pallas_call(
        body,
        grid=(1,),
        in_specs=[pl.BlockSpec((NC * NS, w), lambda i: (0, 0))],
        out_specs=pl.BlockSpec((n, 1), lambda i: (0, 0)),
        out_shape=jax.ShapeDtypeStruct((n, 1), jnp.float32),
    )(degp)


def _tc_first(x, w1, dinv, bn):
    n, d_in = x.shape
    nb = n // bn

    def body(x_ref, w_ref, dinv_ref, o_ref):
        h = jnp.dot(x_ref[...], w_ref[...], preferred_element_type=jnp.float32)
        o_ref[...] = h * dinv_ref[...]

    return pl.pallas_call(
        body,
        grid=(nb, NC),
        in_specs=[
            pl.BlockSpec((bn, d_in), lambda i, c: (i, 0)),
            pl.BlockSpec((d_in, 128), lambda i, c: (0, c)),
            pl.BlockSpec((bn, 1), lambda i, c: (i, 0)),
        ],
        out_specs=pl.BlockSpec((bn, 128), lambda i, c, _nb=nb: (c * _nb + i, 0)),
        out_shape=jax.ShapeDtypeStruct((NC * n, 128), jnp.float32),
    )(x, w1, dinv)


def _tc_mid(sacc, hprev, dinv, w, n, bn):
    nb = n // bn

    def body(sa_ref, sb_ref, ha_ref, hb_ref, w_ref, dinv_ref, o_ref):
        dv = dinv_ref[...]
        z0 = jnp.maximum((sa_ref[...] + ha_ref[...]) * dv, 0.0)
        z1 = jnp.maximum((sb_ref[...] + hb_ref[...]) * dv, 0.0)
        h = (jnp.dot(z0, w_ref[0:128, :], preferred_element_type=jnp.float32)
             + jnp.dot(z1, w_ref[128:256, :], preferred_element_type=jnp.float32))
        o_ref[...] = h * dv

    half_a = pl.BlockSpec((bn, 128), lambda i, c: (i, 0))
    half_b = pl.BlockSpec((bn, 128), lambda i, c, _nb=nb: (_nb + i, 0))
    return pl.pallas_call(
        body,
        grid=(nb, NC),
        in_specs=[
            half_a, half_b, half_a, half_b,
            pl.BlockSpec((256, 128), lambda i, c: (0, c)),
            pl.BlockSpec((bn, 1), lambda i, c: (i, 0)),
        ],
        out_specs=pl.BlockSpec((bn, 128), lambda i, c, _nb=nb: (c * _nb + i, 0)),
        out_shape=jax.ShapeDtypeStruct((NC * n, 128), jnp.float32),
    )(sacc, sacc, hprev, hprev, w, dinv)


def _tc_final(sacc, hprev, dinv, n, bn):
    nb = n // bn

    def body(sa_ref, sb_ref, ha_ref, hb_ref, dinv_ref, o_ref):
        dv = dinv_ref[...]
        o_ref[:, 0:128] = (sa_ref[...] + ha_ref[...]) * dv
        o_ref[:, 128:256] = (sb_ref[...] + hb_ref[...]) * dv

    half_a = pl.BlockSpec((bn, 128), lambda i: (i, 0))
    half_b = pl.BlockSpec((bn, 128), lambda i, _nb=nb: (_nb + i, 0))
    return pl.pallas_call(
        body,
        grid=(nb,),
        in_specs=[
            half_a, half_b, half_a, half_b,
            pl.BlockSpec((bn, 1), lambda i: (i, 0)),
        ],
        out_specs=pl.BlockSpec((bn, 256), lambda i: (i, 0)),
        out_shape=jax.ShapeDtypeStruct((n, 256), jnp.float32),
    )(sacc, sacc, hprev, hprev, dinv)


def kernel(x, edge_index, W1, W2, W3):
    n = x.shape[0]
    e = edge_index.shape[1]
    ei = edge_index.astype(jnp.int32)
    src, dst = ei[0], ei[1]

    quantum = NC * NS * CHUNK * BLK
    ep = _cdiv(e, quantum) * quantum
    pad = ep - e
    src_p = jnp.concatenate([src, jnp.zeros((pad,), jnp.int32)])
    dst_p = jnp.concatenate([dst, jnp.full((pad,), n, jnp.int32)])
    src2 = jnp.stack([src_p, src_p + n]).reshape(NC, ep // CHUNK, CHUNK)
    dst2d = dst_p.reshape(ep // CHUNK, CHUNK)

    bn = 1000
    degp = _sc_deg(dst2d, n)
    degp = degp.reshape(degp.shape[0], -1)
    dinv = _tc_dinv(degp, n)

    h1 = _tc_first(x, W1, dinv, bn)
    s1 = _sc_scatter(h1, src2, dst2d, n)
    h2 = _tc_mid(s1, h1, dinv, W2, n, bn)
    s2 = _sc_scatter(h2, src2, dst2d, n)
    h3 = _tc_mid(s2, h2, dinv, W3, n, bn)
    s3 = _sc_scatter(h3, src2, dst2d, n)
    return _tc_final(s3, h3, dinv, n, bn)


# speedup vs baseline: 8.0854x; 1.1333x over previous
import functools

import jax
import jax.numpy as jnp
from jax import lax
from jax.experimental import pallas as pl
from jax.experimental.pallas import tpu as pltpu
from jax.experimental.pallas import tpu_sc as plsc

NC = 2
NS = 16
CHUNK = 128
BLK = 4


def _cdiv(a, b):
    return (a + b - 1) // b


def _sc_deg(dst2d, n_nodes):
    n_chunks = dst2d.shape[0]
    ct = n_chunks // (NC * NS)
    hr = _cdiv(n_nodes + 1, 128)

    mesh = plsc.VectorSubcoreMesh(core_axis_name="c", subcore_axis_name="s", num_cores=NC, num_subcores=NS)

    @functools.partial(
        pl.kernel,
        out_type=jax.ShapeDtypeStruct((NC * NS, hr, 128), jnp.float32),
        mesh=mesh,
        compiler_params=pltpu.CompilerParams(needs_layout_passes=False),
        scratch_types=[
            pltpu.VMEM((hr, 128), jnp.float32),
            pltpu.VMEM((ct, CHUNK), jnp.int32),
        ],
    )
    def deg_kernel(dst_hbm, deg_out, hist, dbuf):
        c = lax.axis_index("c")
        s = lax.axis_index("s")
        wid = s * NC + c

        def zero_body(i, _):
            hist[i // 8, pl.ds((i % 8) * 16, 16)] = jnp.zeros((16,), jnp.float32)
            return _

        lax.fori_loop(0, hr * 8, zero_body, 0)

        pltpu.sync_copy(dst_hbm.at[pl.ds(wid * ct, ct)], dbuf)

        ones = jnp.ones((16,), jnp.float32)

        def acc_body(i, _):
            row = i // (CHUNK // 16)
            grp = i % (CHUNK // 16)
            idx = dbuf[row, pl.ds(grp * 16, 16)]
            plsc.addupdate_scatter(
                hist,
                [lax.shift_right_logical(idx, 7), lax.bitwise_and(idx, 127)],
                ones)
            return _

        lax.fori_loop(0, ct * (CHUNK // 16), acc_body, 0)

        pltpu.sync_copy(hist, deg_out.at[wid])

    return deg_kernel(dst2d)


def _sc_scatter(h_stacked, comb, n_nodes):
    n_chunks = comb.shape[1] // 2
    ct = n_chunks // NS
    iblk = 8
    gct = ct // iblk
    acc_r = _cdiv(n_nodes + 8, NS * 8) * NS * 8
    zr = acc_r // NS
    wr_a = _cdiv(n_nodes // NS, 8) * 8
    wr_b = n_nodes - (NS - 1) * wr_a

    mesh = plsc.VectorSubcoreMesh(core_axis_name="c", subcore_axis_name="s", num_cores=NC, num_subcores=NS)

    @functools.partial(
        pl.kernel,
        out_type=jax.ShapeDtypeStruct((NC * n_nodes, 128), jnp.float32),
        mesh=mesh,
        scratch_types=[
            pltpu.VMEM_SHARED((acc_r, 128), jnp.float32),
            pltpu.VMEM((2 * CHUNK, 128), jnp.float32),
            pltpu.VMEM((2, 2 * iblk, CHUNK), jnp.int32),
            pltpu.SemaphoreType.DMA,
            pltpu.SemaphoreType.DMA,
        ],
    )
    def scat_kernel(h_hbm, comb_hbm, out_hbm, acc, rowbuf, idxb, sem0, sem1):
        c = lax.axis_index("c")
        s = lax.axis_index("s")
        sems = (sem0, sem1)

        def buf(a):
            return rowbuf.at[pl.ds(a * CHUNK, CHUNK)]

        def fire_g(a, gslot, row):
            pltpu.async_copy(h_hbm.at[idxb.at[gslot, row]], buf(a), sems[a])

        def wait_g(a):
            pltpu.make_async_copy(h_hbm.at[pl.ds(0, CHUNK)], buf(a),
                                  sems[a]).wait()

        def load_group(g, gslot):
            off = pl.multiple_of((s * gct + g) * 2 * iblk, 8)
            pltpu.sync_copy(comb_hbm.at[c, pl.ds(off, 2 * iblk)],
                            idxb.at[gslot])

        def zb_body(i, _):
            rowbuf[i // 8, pl.ds((i % 8) * 16, 16)] = jnp.zeros((16,), jnp.float32)
            return _

        lax.fori_loop(0, CHUNK * 8, zb_body, 0)

        base = pl.multiple_of(s * zr, 8)
        nfull, rem = zr // CHUNK, zr % CHUNK
        for k in range(nfull):
            pltpu.sync_copy(rowbuf.at[pl.ds(0, CHUNK)],
                            acc.at[pl.ds(base + k * CHUNK, CHUNK)])
        if rem:
            pltpu.sync_copy(rowbuf.at[pl.ds(0, rem)],
                            acc.at[pl.ds(base + nfull * CHUNK, rem)])
        plsc.subcore_barrier()

        load_group(0, 0)
        fire_g(0, 0, 0)

        def outer(b, carry):
            gb = b % 2

            @pl.when(b < gct - 1)
            def _():
                load_group(b + 1, (b + 1) % 2)

            for j in range(iblk):
                a = j % 2
                wait_g(a)
                if j < iblk - 1:
                    fire_g(1 - a, gb, 2 * (j + 1))
                else:
                    @pl.when(b < gct - 1)
                    def _():
                        fire_g(1 - a, (b + 1) % 2, 0)
                pltpu.sync_copy(buf(a), acc.at[idxb.at[gb, 2 * j + 1]],
                                add=True)
            return carry

        lax.fori_loop(0, gct, outer, 0)

        plsc.subcore_barrier()
        rd = pl.multiple_of(s * wr_a, 8)
        wo = pl.multiple_of(c * n_nodes + s * wr_a, 8)

        @pl.when(s < NS - 1)
        def _():
            pltpu.sync_copy(acc.at[pl.ds(rd, wr_a)], out_hbm.at[pl.ds(wo, wr_a)])

        @pl.when(s == NS - 1)
        def _():
            pltpu.sync_copy(acc.at[pl.ds(rd, wr_b)], out_hbm.at[pl.ds(wo, wr_b)])

    return scat_kernel(h_stacked, comb)


def _tc_dinv(degp, n):
    w = degp.shape[1]

    def body(degp_ref, o_ref):
        deg = jnp.sum(degp_ref[...], axis=0)[:n]
        o_ref[...] = lax.rsqrt(1.0 + deg)[:, None]

    return pl.pallas_call(
        body,
        grid=(1,),
        in_specs=[pl.BlockSpec((NC * NS, w), lambda i: (0, 0))],
        out_specs=pl.BlockSpec((n, 1), lambda i: (0, 0)),
        out_shape=jax.ShapeDtypeStruct((n, 1), jnp.float32),
    )(degp)


def _tc_first(x, w1, dinv, bn):
    n, d_in = x.shape
    nb = n // bn

    def body(x_ref, w_ref, dinv_ref, o_ref):
        h = jnp.dot(x_ref[...], w_ref[...], preferred_element_type=jnp.float32)
        o_ref[...] = h * dinv_ref[...]

    return pl.pallas_call(
        body,
        grid=(nb, NC),
        in_specs=[
            pl.BlockSpec((bn, d_in), lambda i, c: (i, 0)),
            pl.BlockSpec((d_in, 128), lambda i, c: (0, c)),
            pl.BlockSpec((bn, 1), lambda i, c: (i, 0)),
        ],
        out_specs=pl.BlockSpec((bn, 128), lambda i, c, _nb=nb: (c * _nb + i, 0)),
        out_shape=jax.ShapeDtypeStruct((NC * n, 128), jnp.float32),
    )(x, w1, dinv)


def _tc_mid(sacc, hprev, dinv, w, n, bn):
    nb = n // bn

    def body(sa_ref, sb_ref, ha_ref, hb_ref, w_ref, dinv_ref, o_ref):
        dv = dinv_ref[...]
        z0 = jnp.maximum((sa_ref[...] + ha_ref[...]) * dv, 0.0)
        z1 = jnp.maximum((sb_ref[...] + hb_ref[...]) * dv, 0.0)
        h = (jnp.dot(z0, w_ref[0:128, :], preferred_element_type=jnp.float32)
             + jnp.dot(z1, w_ref[128:256, :], preferred_element_type=jnp.float32))
        o_ref[...] = h * dv

    half_a = pl.BlockSpec((bn, 128), lambda i, c: (i, 0))
    half_b = pl.BlockSpec((bn, 128), lambda i, c, _nb=nb: (_nb + i, 0))
    return pl.pallas_call(
        body,
        grid=(nb, NC),
        in_specs=[
            half_a, half_b, half_a, half_b,
            pl.BlockSpec((256, 128), lambda i, c: (0, c)),
            pl.BlockSpec((bn, 1), lambda i, c: (i, 0)),
        ],
        out_specs=pl.BlockSpec((bn, 128), lambda i, c, _nb=nb: (c * _nb + i, 0)),
        out_shape=jax.ShapeDtypeStruct((NC * n, 128), jnp.float32),
    )(sacc, sacc, hprev, hprev, w, dinv)


def _tc_final(sacc, hprev, dinv, n, bn):
    nb = n // bn

    def body(sa_ref, sb_ref, ha_ref, hb_ref, dinv_ref, o_ref):
        dv = dinv_ref[...]
        o_ref[:, 0:128] = (sa_ref[...] + ha_ref[...]) * dv
        o_ref[:, 128:256] = (sb_ref[...] + hb_ref[...]) * dv

    half_a = pl.BlockSpec((bn, 128), lambda i: (i, 0))
    half_b = pl.BlockSpec((bn, 128), lambda i, _nb=nb: (_nb + i, 0))
    return pl.pallas_call(
        body,
        grid=(nb,),
        in_specs=[
            half_a, half_b, half_a, half_b,
            pl.BlockSpec((bn, 1), lambda i: (i, 0)),
        ],
        out_specs=pl.BlockSpec((bn, 256), lambda i: (i, 0)),
        out_shape=jax.ShapeDtypeStruct((n, 256), jnp.float32),
    )(sacc, sacc, hprev, hprev, dinv)


def kernel(x, edge_index, W1, W2, W3):
    n = x.shape[0]
    e = edge_index.shape[1]
    ei = edge_index.astype(jnp.int32)
    src, dst = ei[0], ei[1]

    quantum = NC * NS * CHUNK * BLK
    ep = _cdiv(e, quantum) * quantum
    pad = ep - e
    src_p = jnp.concatenate([src, jnp.zeros((pad,), jnp.int32)])
    dst_p = jnp.concatenate([dst, jnp.full((pad,), n, jnp.int32)])
    src2 = jnp.stack([src_p, src_p + n]).reshape(NC, ep // CHUNK, CHUNK)
    dst2d = dst_p.reshape(ep // CHUNK, CHUNK)
    comb = jnp.stack(
        [src2, jnp.broadcast_to(dst2d, src2.shape)], axis=2,
    ).reshape(NC, 2 * (ep // CHUNK), CHUNK)

    bn = 1000
    degp = _sc_deg(dst2d, n)
    degp = degp.reshape(degp.shape[0], -1)
    dinv = _tc_dinv(degp, n)

    h1 = _tc_first(x, W1, dinv, bn)
    s1 = _sc_scatter(h1, comb, n)
    h2 = _tc_mid(s1, h1, dinv, W2, n, bn)
    s2 = _sc_scatter(h2, comb, n)
    h3 = _tc_mid(s2, h2, dinv, W3, n, bn)
    s3 = _sc_scatter(h3, comb, n)
    return _tc_final(s3, h3, dinv, n, bn)


# speedup vs baseline: 8.1362x; 1.0063x over previous
import functools

import jax
import jax.numpy as jnp
from jax import lax
from jax.experimental import pallas as pl
from jax.experimental.pallas import tpu as pltpu
from jax.experimental.pallas import tpu_sc as plsc

NC = 2
NS = 16
CHUNK = 128
BLK = 4


def _cdiv(a, b):
    return (a + b - 1) // b


def _sc_deg(dst2d, n_nodes):
    n_chunks = dst2d.shape[0]
    ct = n_chunks // (NC * NS)
    hr = _cdiv(n_nodes + 1, 128)

    mesh = plsc.VectorSubcoreMesh(core_axis_name="c", subcore_axis_name="s", num_cores=NC, num_subcores=NS)

    @functools.partial(
        pl.kernel,
        out_type=jax.ShapeDtypeStruct((NC * NS, hr, 128), jnp.float32),
        mesh=mesh,
        compiler_params=pltpu.CompilerParams(needs_layout_passes=False),
        scratch_types=[
            pltpu.VMEM((hr, 128), jnp.float32),
            pltpu.VMEM((ct, CHUNK), jnp.int32),
        ],
    )
    def deg_kernel(dst_hbm, deg_out, hist, dbuf):
        c = lax.axis_index("c")
        s = lax.axis_index("s")
        wid = s * NC + c

        def zero_body(i, _):
            hist[i // 8, pl.ds((i % 8) * 16, 16)] = jnp.zeros((16,), jnp.float32)
            return _

        lax.fori_loop(0, hr * 8, zero_body, 0)

        pltpu.sync_copy(dst_hbm.at[pl.ds(wid * ct, ct)], dbuf)

        ones = jnp.ones((16,), jnp.float32)

        def acc_body(i, _):
            row = i // (CHUNK // 16)
            grp = i % (CHUNK // 16)
            idx = dbuf[row, pl.ds(grp * 16, 16)]
            plsc.addupdate_scatter(
                hist,
                [lax.shift_right_logical(idx, 7), lax.bitwise_and(idx, 127)],
                ones)
            return _

        lax.fori_loop(0, ct * (CHUNK // 16), acc_body, 0)

        pltpu.sync_copy(hist, deg_out.at[wid])

    return deg_kernel(dst2d)


def _sc_scatter(h_stacked, comb, n_nodes):
    n_chunks = comb.shape[1] // 2
    ct = n_chunks // NS
    iblk = 8
    gct = ct // iblk
    acc_r = _cdiv(n_nodes + 8, NS * 8) * NS * 8
    zr = acc_r // NS
    wr_a = _cdiv(n_nodes // NS, 8) * 8
    wr_b = n_nodes - (NS - 1) * wr_a

    mesh = plsc.VectorSubcoreMesh(core_axis_name="c", subcore_axis_name="s", num_cores=NC, num_subcores=NS)

    @functools.partial(
        pl.kernel,
        out_type=jax.ShapeDtypeStruct((NC * n_nodes, 128), jnp.float32),
        mesh=mesh,
        scratch_types=[
            pltpu.VMEM_SHARED((acc_r, 128), jnp.float32),
            pltpu.VMEM((2 * CHUNK, 128), jnp.float32),
            pltpu.VMEM((3, 2 * iblk, CHUNK), jnp.int32),
            pltpu.SemaphoreType.DMA,
            pltpu.SemaphoreType.DMA,
            pltpu.SemaphoreType.DMA,
            pltpu.SemaphoreType.DMA,
        ],
    )
    def scat_kernel(h_hbm, comb_hbm, out_hbm, acc, rowbuf, idxb,
                    gsem0, gsem1, ssem0, ssem1):
        c = lax.axis_index("c")
        s = lax.axis_index("s")
        gsems = (gsem0, gsem1)
        ssems = (ssem0, ssem1)

        def buf(a):
            return rowbuf.at[pl.ds(a * CHUNK, CHUNK)]

        def fire_g(a, gslot, row):
            pltpu.async_copy(h_hbm.at[idxb.at[gslot, row]], buf(a), gsems[a])

        def wait_g(a):
            pltpu.make_async_copy(h_hbm.at[pl.ds(0, CHUNK)], buf(a),
                                  gsems[a]).wait()

        def fire_s(a, gslot, row):
            pltpu.async_copy(buf(a), acc.at[idxb.at[gslot, row]], ssems[a],
                             add=True)

        def wait_s(a):
            pltpu.make_async_copy(h_hbm.at[pl.ds(0, CHUNK)],
                                  acc.at[pl.ds(0, CHUNK)], ssems[a]).wait()

        def load_group(g, gslot):
            off = pl.multiple_of((s * gct + g) * 2 * iblk, 8)
            pltpu.sync_copy(comb_hbm.at[c, pl.ds(off, 2 * iblk)],
                            idxb.at[gslot])

        def zb_body(i, _):
            rowbuf[i // 8, pl.ds((i % 8) * 16, 16)] = jnp.zeros((16,), jnp.float32)
            return _

        lax.fori_loop(0, CHUNK * 8, zb_body, 0)

        base = pl.multiple_of(s * zr, 8)
        nfull, rem = zr // CHUNK, zr % CHUNK
        for k in range(nfull):
            pltpu.sync_copy(rowbuf.at[pl.ds(0, CHUNK)],
                            acc.at[pl.ds(base + k * CHUNK, CHUNK)])
        if rem:
            pltpu.sync_copy(rowbuf.at[pl.ds(0, rem)],
                            acc.at[pl.ds(base + nfull * CHUNK, rem)])
        plsc.subcore_barrier()

        load_group(0, 0)
        fire_g(0, 0, 0)

        def outer(b, carry):
            gb = b % 3

            @pl.when(b < gct - 1)
            def _():
                load_group(b + 1, (b + 1) % 3)

            for j in range(iblk):
                a = j % 2
                wait_g(a)
                fire_s(a, gb, 2 * j + 1)
                if j < iblk - 1:
                    if j == 0:
                        @pl.when(b > 0)
                        def _():
                            wait_s(1 - a)
                    else:
                        wait_s(1 - a)
                    fire_g(1 - a, gb, 2 * (j + 1))
                else:
                    @pl.when(b < gct - 1)
                    def _():
                        wait_s(1 - a)
                        fire_g(1 - a, (b + 1) % 3, 0)
            return carry

        lax.fori_loop(0, gct, outer, 0)

        wait_s(0)
        wait_s(1)
        plsc.subcore_barrier()
        rd = pl.multiple_of(s * wr_a, 8)
        wo = pl.multiple_of(c * n_nodes + s * wr_a, 8)

        @pl.when(s < NS - 1)
        def _():
            pltpu.sync_copy(acc.at[pl.ds(rd, wr_a)], out_hbm.at[pl.ds(wo, wr_a)])

        @pl.when(s == NS - 1)
        def _():
            pltpu.sync_copy(acc.at[pl.ds(rd, wr_b)], out_hbm.at[pl.ds(wo, wr_b)])

    return scat_kernel(h_stacked, comb)


def _tc_dinv(degp, n):
    w = degp.shape[1]

    def body(degp_ref, o_ref):
        deg = jnp.sum(degp_ref[...], axis=0)[:n]
        o_ref[...] = lax.rsqrt(1.0 + deg)[:, None]

    return pl.pallas_call(
        body,
        grid=(1,),
        in_specs=[pl.BlockSpec((NC * NS, w), lambda i: (0, 0))],
        out_specs=pl.BlockSpec((n, 1), lambda i: (0, 0)),
        out_shape=jax.ShapeDtypeStruct((n, 1), jnp.float32),
    )(degp)


def _tc_first(x, w1, dinv, bn):
    n, d_in = x.shape
    nb = n // bn

    def body(x_ref, w_ref, dinv_ref, o_ref):
        h = jnp.dot(x_ref[...], w_ref[...], preferred_element_type=jnp.float32)
        o_ref[...] = h * dinv_ref[...]

    return pl.pallas_call(
        body,
        grid=(nb, NC),
        in_specs=[
            pl.BlockSpec((bn, d_in), lambda i, c: (i, 0)),
            pl.BlockSpec((d_in, 128), lambda i, c: (0, c)),
            pl.BlockSpec((bn, 1), lambda i, c: (i, 0)),
        ],
        out_specs=pl.BlockSpec((bn, 128), lambda i, c, _nb=nb: (c * _nb + i, 0)),
        out_shape=jax.ShapeDtypeStruct((NC * n, 128), jnp.float32),
    )(x, w1, dinv)


def _tc_mid(sacc, hprev, dinv, w, n, bn):
    nb = n // bn

    def body(sa_ref, sb_ref, ha_ref, hb_ref, w_ref, dinv_ref, o_ref):
        dv = dinv_ref[...]
        z0 = jnp.maximum((sa_ref[...] + ha_ref[...]) * dv, 0.0)
        z1 = jnp.maximum((sb_ref[...] + hb_ref[...]) * dv, 0.0)
        h = (jnp.dot(z0, w_ref[0:128, :], preferred_element_type=jnp.float32)
             + jnp.dot(z1, w_ref[128:256, :], preferred_element_type=jnp.float32))
        o_ref[...] = h * dv

    half_a = pl.BlockSpec((bn, 128), lambda i, c: (i, 0))
    half_b = pl.BlockSpec((bn, 128), lambda i, c, _nb=nb: (_nb + i, 0))
    return pl.pallas_call(
        body,
        grid=(nb, NC),
        in_specs=[
            half_a, half_b, half_a, half_b,
            pl.BlockSpec((256, 128), lambda i, c: (0, c)),
            pl.BlockSpec((bn, 1), lambda i, c: (i, 0)),
        ],
        out_specs=pl.BlockSpec((bn, 128), lambda i, c, _nb=nb: (c * _nb + i, 0)),
        out_shape=jax.ShapeDtypeStruct((NC * n, 128), jnp.float32),
    )(sacc, sacc, hprev, hprev, w, dinv)


def _tc_final(sacc, hprev, dinv, n, bn):
    nb = n // bn

    def body(sa_ref, sb_ref, ha_ref, hb_ref, dinv_ref, o_ref):
        dv = dinv_ref[...]
        o_ref[:, 0:128] = (sa_ref[...] + ha_ref[...]) * dv
        o_ref[:, 128:256] = (sb_ref[...] + hb_ref[...]) * dv

    half_a = pl.BlockSpec((bn, 128), lambda i: (i, 0))
    half_b = pl.BlockSpec((bn, 128), lambda i, _nb=nb: (_nb + i, 0))
    return pl.pallas_call(
        body,
        grid=(nb,),
        in_specs=[
            half_a, half_b, half_a, half_b,
            pl.BlockSpec((bn, 1), lambda i: (i, 0)),
        ],
        out_specs=pl.BlockSpec((bn, 256), lambda i: (i, 0)),
        out_shape=jax.ShapeDtypeStruct((n, 256), jnp.float32),
    )(sacc, sacc, hprev, hprev, dinv)


def kernel(x, edge_index, W1, W2, W3):
    n = x.shape[0]
    e = edge_index.shape[1]
    ei = edge_index.astype(jnp.int32)
    src, dst = ei[0], ei[1]

    quantum = NC * NS * CHUNK * BLK
    ep = _cdiv(e, quantum) * quantum
    pad = ep - e
    src_p = jnp.concatenate([src, jnp.zeros((pad,), jnp.int32)])
    dst_p = jnp.concatenate([dst, jnp.full((pad,), n, jnp.int32)])
    src2 = jnp.stack([src_p, src_p + n]).reshape(NC, ep // CHUNK, CHUNK)
    dst2d = dst_p.reshape(ep // CHUNK, CHUNK)
    comb = jnp.stack(
        [src2, jnp.broadcast_to(dst2d, src2.shape)], axis=2,
    ).reshape(NC, 2 * (ep // CHUNK), CHUNK)

    bn = 1000
    degp = _sc_deg(dst2d, n)
    degp = degp.reshape(degp.shape[0], -1)
    dinv = _tc_dinv(degp, n)

    h1 = _tc_first(x, W1, dinv, bn)
    s1 = _sc_scatter(h1, comb, n)
    h2 = _tc_mid(s1, h1, dinv, W2, n, bn)
    s2 = _sc_scatter(h2, comb, n)
    h3 = _tc_mid(s2, h2, dinv, W3, n, bn)
    s3 = _sc_scatter(h3, comb, n)
    return _tc_final(s3, h3, dinv, n, bn)


# speedup vs baseline: 8.5872x; 1.0554x over previous
import functools

import jax
import jax.numpy as jnp
from jax import lax
from jax.experimental import pallas as pl
from jax.experimental.pallas import tpu as pltpu
from jax.experimental.pallas import tpu_sc as plsc

NC = 2
NS = 16
CHUNK = 128
BLK = 4


def _cdiv(a, b):
    return (a + b - 1) // b


def _sc_deg(dst2d, n_nodes):
    n_chunks = dst2d.shape[0]
    ct = n_chunks // (NC * NS)
    hr = _cdiv(n_nodes + 1, 128)

    mesh = plsc.VectorSubcoreMesh(core_axis_name="c", subcore_axis_name="s", num_cores=NC, num_subcores=NS)

    @functools.partial(
        pl.kernel,
        out_type=jax.ShapeDtypeStruct((NC * NS, hr, 128), jnp.float32),
        mesh=mesh,
        compiler_params=pltpu.CompilerParams(needs_layout_passes=False),
        scratch_types=[
            pltpu.VMEM((hr, 128), jnp.float32),
            pltpu.VMEM((ct, CHUNK), jnp.int32),
        ],
    )
    def deg_kernel(dst_hbm, deg_out, hist, dbuf):
        c = lax.axis_index("c")
        s = lax.axis_index("s")
        wid = s * NC + c

        def zero_body(i, _):
            hist[i // 8, pl.ds((i % 8) * 16, 16)] = jnp.zeros((16,), jnp.float32)
            return _

        lax.fori_loop(0, hr * 8, zero_body, 0)

        pltpu.sync_copy(dst_hbm.at[pl.ds(wid * ct, ct)], dbuf)

        ones = jnp.ones((16,), jnp.float32)

        def acc_body(i, _):
            row = i // (CHUNK // 16)
            grp = i % (CHUNK // 16)
            idx = dbuf[row, pl.ds(grp * 16, 16)]
            plsc.addupdate_scatter(
                hist,
                [lax.shift_right_logical(idx, 7), lax.bitwise_and(idx, 127)],
                ones)
            return _

        lax.fori_loop(0, ct * (CHUNK // 16), acc_body, 0)

        pltpu.sync_copy(hist, deg_out.at[wid])

    return deg_kernel(dst2d)


def _sc_scatter(h_stacked, comb, n_nodes):
    n_chunks = comb.shape[1] // 2
    ct = n_chunks // NS
    iblk = 8
    gct = ct // iblk
    acc_r = _cdiv(n_nodes + 8, NS * 8) * NS * 8
    zr = acc_r // NS
    wr_a = _cdiv(n_nodes // NS, 8) * 8
    wr_b = n_nodes - (NS - 1) * wr_a

    mesh = plsc.VectorSubcoreMesh(core_axis_name="c", subcore_axis_name="s", num_cores=NC, num_subcores=NS)

    @functools.partial(
        pl.kernel,
        out_type=jax.ShapeDtypeStruct((NC * n_nodes, 128), jnp.float32),
        mesh=mesh,
        scratch_types=[
            pltpu.VMEM_SHARED((acc_r, 128), jnp.float32),
            pltpu.VMEM((2 * CHUNK, 128), jnp.float32),
            pltpu.VMEM((3, 2 * iblk, CHUNK), jnp.int32),
            pltpu.SemaphoreType.DMA,
            pltpu.SemaphoreType.DMA,
            pltpu.SemaphoreType.DMA,
            pltpu.SemaphoreType.DMA,
        ],
    )
    def scat_kernel(h_hbm, comb_hbm, out_hbm, acc, rowbuf, idxb,
                    gsem0, gsem1, ssem0, ssem1):
        c = lax.axis_index("c")
        s = lax.axis_index("s")
        gsems = (gsem0, gsem1)
        ssems = (ssem0, ssem1)

        def buf(a):
            return rowbuf.at[pl.ds(a * CHUNK, CHUNK)]

        def fire_g(a, gslot, row):
            pltpu.async_copy(h_hbm.at[idxb.at[gslot, row]], buf(a), gsems[a])

        def wait_g(a):
            pltpu.make_async_copy(h_hbm.at[pl.ds(0, CHUNK)], buf(a),
                                  gsems[a]).wait()

        def fire_s(a, gslot, row):
            pltpu.async_copy(buf(a), acc.at[idxb.at[gslot, row]], ssems[a],
                             add=True)

        def wait_s(a):
            pltpu.make_async_copy(h_hbm.at[pl.ds(0, CHUNK)],
                                  acc.at[pl.ds(0, CHUNK)], ssems[a]).wait()

        def load_group(g, gslot):
            off = pl.multiple_of((s * gct + g) * 2 * iblk, 8)
            pltpu.sync_copy(comb_hbm.at[c, pl.ds(off, 2 * iblk)],
                            idxb.at[gslot])

        def zb_body(i, _):
            rowbuf[i // 8, pl.ds((i % 8) * 16, 16)] = jnp.zeros((16,), jnp.float32)
            return _

        lax.fori_loop(0, CHUNK * 8, zb_body, 0)

        base = pl.multiple_of(s * zr, 8)
        nfull, rem = zr // CHUNK, zr % CHUNK
        for k in range(nfull):
            pltpu.sync_copy(rowbuf.at[pl.ds(0, CHUNK)],
                            acc.at[pl.ds(base + k * CHUNK, CHUNK)])
        if rem:
            pltpu.sync_copy(rowbuf.at[pl.ds(0, rem)],
                            acc.at[pl.ds(base + nfull * CHUNK, rem)])
        plsc.subcore_barrier()

        load_group(0, 0)
        fire_g(0, 0, 0)

        def outer(b, carry):
            gb = b % 3

            @pl.when(b < gct - 1)
            def _():
                load_group(b + 1, (b + 1) % 3)

            for j in range(iblk):
                a = j % 2
                if j < iblk - 1:
                    if j == 0:
                        @pl.when(b > 0)
                        def _():
                            wait_s(1 - a)
                    else:
                        wait_s(1 - a)
                    fire_g(1 - a, gb, 2 * (j + 1))
                else:
                    @pl.when(b < gct - 1)
                    def _():
                        wait_s(1 - a)
                        fire_g(1 - a, (b + 1) % 3, 0)
                wait_g(a)
                fire_s(a, gb, 2 * j + 1)
            return carry

        lax.fori_loop(0, gct, outer, 0)

        wait_s(0)
        wait_s(1)
        plsc.subcore_barrier()
        rd = pl.multiple_of(s * wr_a, 8)
        wo = pl.multiple_of(c * n_nodes + s * wr_a, 8)

        @pl.when(s < NS - 1)
        def _():
            pltpu.sync_copy(acc.at[pl.ds(rd, wr_a)], out_hbm.at[pl.ds(wo, wr_a)])

        @pl.when(s == NS - 1)
        def _():
            pltpu.sync_copy(acc.at[pl.ds(rd, wr_b)], out_hbm.at[pl.ds(wo, wr_b)])

    return scat_kernel(h_stacked, comb)


def _tc_dinv(degp, n):
    w = degp.shape[1]

    def body(degp_ref, o_ref):
        deg = jnp.sum(degp_ref[...], axis=0)[:n]
        o_ref[...] = lax.rsqrt(1.0 + deg)[:, None]

    return pl.pallas_call(
        body,
        grid=(1,),
        in_specs=[pl.BlockSpec((NC * NS, w), lambda i: (0, 0))],
        out_specs=pl.BlockSpec((n, 1), lambda i: (0, 0)),
        out_shape=jax.ShapeDtypeStruct((n, 1), jnp.float32),
    )(degp)


def _tc_first(x, w1, dinv, bn):
    n, d_in = x.shape
    nb = n // bn

    def body(x_ref, w_ref, dinv_ref, o_ref):
        h = jnp.dot(x_ref[...], w_ref[...], preferred_element_type=jnp.float32)
        o_ref[...] = h * dinv_ref[...]

    return pl.pallas_call(
        body,
        grid=(nb, NC),
        in_specs=[
            pl.BlockSpec((bn, d_in), lambda i, c: (i, 0)),
            pl.BlockSpec((d_in, 128), lambda i, c: (0, c)),
            pl.BlockSpec((bn, 1), lambda i, c: (i, 0)),
        ],
        out_specs=pl.BlockSpec((bn, 128), lambda i, c, _nb=nb: (c * _nb + i, 0)),
        out_shape=jax.ShapeDtypeStruct((NC * n, 128), jnp.float32),
    )(x, w1, dinv)


def _tc_mid(sacc, hprev, dinv, w, n, bn):
    nb = n // bn

    def body(sa_ref, sb_ref, ha_ref, hb_ref, w_ref, dinv_ref, o_ref):
        dv = dinv_ref[...]
        z0 = jnp.maximum((sa_ref[...] + ha_ref[...]) * dv, 0.0)
        z1 = jnp.maximum((sb_ref[...] + hb_ref[...]) * dv, 0.0)
        h = (jnp.dot(z0, w_ref[0:128, :], preferred_element_type=jnp.float32)
             + jnp.dot(z1, w_ref[128:256, :], preferred_element_type=jnp.float32))
        o_ref[...] = h * dv

    half_a = pl.BlockSpec((bn, 128), lambda i, c: (i, 0))
    half_b = pl.BlockSpec((bn, 128), lambda i, c, _nb=nb: (_nb + i, 0))
    return pl.pallas_call(
        body,
        grid=(nb, NC),
        in_specs=[
            half_a, half_b, half_a, half_b,
            pl.BlockSpec((256, 128), lambda i, c: (0, c)),
            pl.BlockSpec((bn, 1), lambda i, c: (i, 0)),
        ],
        out_specs=pl.BlockSpec((bn, 128), lambda i, c, _nb=nb: (c * _nb + i, 0)),
        out_shape=jax.ShapeDtypeStruct((NC * n, 128), jnp.float32),
    )(sacc, sacc, hprev, hprev, w, dinv)


def _tc_final(sacc, hprev, dinv, n, bn):
    nb = n // bn

    def body(sa_ref, sb_ref, ha_ref, hb_ref, dinv_ref, o_ref):
        dv = dinv_ref[...]
        o_ref[:, 0:128] = (sa_ref[...] + ha_ref[...]) * dv
        o_ref[:, 128:256] = (sb_ref[...] + hb_ref[...]) * dv

    half_a = pl.BlockSpec((bn, 128), lambda i: (i, 0))
    half_b = pl.BlockSpec((bn, 128), lambda i, _nb=nb: (_nb + i, 0))
    return pl.pallas_call(
        body,
        grid=(nb,),
        in_specs=[
            half_a, half_b, half_a, half_b,
            pl.BlockSpec((bn, 1), lambda i: (i, 0)),
        ],
        out_specs=pl.BlockSpec((bn, 256), lambda i: (i, 0)),
        out_shape=jax.ShapeDtypeStruct((n, 256), jnp.float32),
    )(sacc, sacc, hprev, hprev, dinv)


def kernel(x, edge_index, W1, W2, W3):
    n = x.shape[0]
    e = edge_index.shape[1]
    ei = edge_index.astype(jnp.int32)
    src, dst = ei[0], ei[1]

    quantum = NC * NS * CHUNK * BLK
    ep = _cdiv(e, quantum) * quantum
    pad = ep - e
    src_p = jnp.concatenate([src, jnp.zeros((pad,), jnp.int32)])
    dst_p = jnp.concatenate([dst, jnp.full((pad,), n, jnp.int32)])
    src2 = jnp.stack([src_p, src_p + n]).reshape(NC, ep // CHUNK, CHUNK)
    dst2d = dst_p.reshape(ep // CHUNK, CHUNK)
    comb = jnp.stack(
        [src2, jnp.broadcast_to(dst2d, src2.shape)], axis=2,
    ).reshape(NC, 2 * (ep // CHUNK), CHUNK)

    bn = 1000
    degp = _sc_deg(dst2d, n)
    degp = degp.reshape(degp.shape[0], -1)
    dinv = _tc_dinv(degp, n)

    h1 = _tc_first(x, W1, dinv, bn)
    s1 = _sc_scatter(h1, comb, n)
    h2 = _tc_mid(s1, h1, dinv, W2, n, bn)
    s2 = _sc_scatter(h2, comb, n)
    h3 = _tc_mid(s2, h2, dinv, W3, n, bn)
    s3 = _sc_scatter(h3, comb, n)
    return _tc_final(s3, h3, dinv, n, bn)
